# baseline (speedup 1.0000x reference)
import functools

import jax
import jax.numpy as jnp
from jax import lax
from jax.experimental import pallas as pl
from jax.experimental.pallas import tpu as pltpu
from jax.experimental.pallas import tpu_sc as plsc

DIM = 64
SCALE = DIM ** (-0.5)

_NC = 2
_NS = 16
_NW = _NC * _NS
_IB = 128


def _make_gather(n_seq, seq_len, d):
    n_blocks = n_seq // _IB
    assert n_blocks % _NW == 0 and d % 16 == 0
    blocks_per_w = n_blocks // _NW
    n_units = blocks_per_w * seq_len
    assert n_units % 4 == 0
    mesh = plsc.VectorSubcoreMesh(core_axis_name="c", subcore_axis_name="s")

    @functools.partial(
        pl.kernel,
        mesh=mesh,
        out_type=jax.ShapeDtypeStruct(
            (seq_len, d // 8, n_blocks, 8, _IB), jnp.float32
        ),
        scratch_types=[
            *[pltpu.VMEM((_IB,), jnp.int32) for _ in range(4)],
            *[pltpu.VMEM((_IB, d), jnp.float32) for _ in range(4)],
            *[pltpu.VMEM((d // 8, 8, _IB + 1), jnp.float32) for _ in range(2)],
            *[pltpu.SemaphoreType.DMA for _ in range(10)],
        ],
        compiler_params=pltpu.CompilerParams(
            use_tc_tiling_on_sc=False, needs_layout_passes=False
        ),
    )
    def gather(table_hbm, idxt_hbm, out_hbm, *scratch):
        ibufs = scratch[0:4]
        gbufs = scratch[4:8]
        tbufs = scratch[8:10]
        isems = scratch[10:14]
        gsems = scratch[14:18]
        ssems = scratch[18:20]

        wid = lax.axis_index("s") * _NC + lax.axis_index("c")

        iota = lax.iota(jnp.int32, 16)
        r_v = lax.rem(iota, 8)
        tk_base = lax.div(iota, 8)
        zeros_v = jnp.zeros((16,), jnp.int32)
        row_stride = _IB + 1
        bases = [
            [
                (tk_base + 2 * q) * (8 * row_stride) + r_v * row_stride + cc
                for cc in range(4)
            ]
            for q in range(d // 16)
        ]

        def unit_ij(u):
            j = lax.rem(u, seq_len)
            g = lax.div(u, seq_len)
            return j, wid * blocks_per_w + g

        def issue_idx(u, b):
            j, blk = unit_ij(u)
            pltpu.async_copy(
                idxt_hbm.at[pl.ds(j * n_seq + blk * _IB, _IB)],
                ibufs[b],
                isems[b],
            )

        def wait_idx(u, b):
            j, blk = unit_ij(u)
            pltpu.make_async_copy(
                idxt_hbm.at[pl.ds(j * n_seq + blk * _IB, _IB)],
                ibufs[b],
                isems[b],
            ).wait()

        def issue_gather(b):
            pltpu.async_copy(table_hbm.at[ibufs[b]], gbufs[b], gsems[b])

        def wait_gather(b):
            pltpu.make_async_copy(
                table_hbm.at[ibufs[b]], gbufs[b], gsems[b]
            ).wait()

        def issue_scatter(u, tb):
            j, blk = unit_ij(u)
            pltpu.async_copy(
                tbufs[tb].at[:, :, pl.ds(0, _IB)],
                out_hbm.at[j, :, blk],
                ssems[tb],
            )

        def wait_scatter(u, tb):
            j, blk = unit_ij(u)
            pltpu.make_async_copy(
                tbufs[tb].at[:, :, pl.ds(0, _IB)],
                out_hbm.at[j, :, blk],
                ssems[tb],
            ).wait()

        def transpose_scale(b, tb):
            gbuf = gbufs[b]
            tbuf = tbufs[tb]

            def row4(ci, carry):
                for cc in range(4):
                    c = ci * 4 + cc
                    c_v = zeros_v + c
                    for q in range(d // 16):
                        v = gbuf[c, pl.ds(16 * q, 16)] * SCALE
                        plsc.store_scatter(
                            tbuf, [tk_base + 2 * q, r_v, c_v], v
                        )
                return carry

            lax.fori_loop(0, _IB // 4, row4, 0)

        for u in range(3):
            issue_idx(u, u)
        for u in range(2):
            wait_idx(u, u)
            issue_gather(u)

        def ring4(k, carry):
            for uu in range(4):
                u = k * 4 + uu

                @pl.when(u + 3 < n_units)
                def _idx():
                    issue_idx(u + 3, (uu + 3) % 4)

                @pl.when(u + 2 < n_units)
                def _gather():
                    wait_idx(u + 2, (uu + 2) % 4)
                    issue_gather((uu + 2) % 4)

                wait_gather(uu)

                @pl.when(u >= 2)
                def _drain():
                    wait_scatter(u - 2, uu % 2)

                transpose_scale(uu, uu % 2)
                issue_scatter(u, uu % 2)
            return carry

        lax.fori_loop(0, n_units // 4, ring4, 0)

        for u in range(n_units - 2, n_units):
            wait_scatter(u, u % 2)

    return gather


@jax.jit
def kernel(x, emb_weight):
    b, s = x.shape
    v, d = emb_weight.shape
    idxt = x.T.reshape(-1).astype(jnp.int32)
    out6 = _make_gather(b, s, d)(emb_weight, idxt)
    out3 = jnp.transpose(out6, (0, 1, 3, 2, 4)).reshape(s, d, b)
    return jnp.transpose(out3, (2, 0, 1))


# speedup vs baseline: 2.5005x; 2.5005x over previous
import functools

import jax
import jax.numpy as jnp
from jax import lax
from jax.experimental import pallas as pl
from jax.experimental.pallas import tpu as pltpu
from jax.experimental.pallas import tpu_sc as plsc

DIM = 64
SCALE = DIM ** (-0.5)

_NC = 2
_NS = 16
_NW = _NC * _NS
_IB = 128


def _make_gather(n_seq, seq_len, d):
    n_blocks = n_seq // _IB
    assert n_blocks % _NW == 0 and d % 16 == 0
    blocks_per_w = n_blocks // _NW
    n_units = blocks_per_w * seq_len
    assert n_units % 4 == 0
    mesh = plsc.VectorSubcoreMesh(core_axis_name="c", subcore_axis_name="s")

    @functools.partial(
        pl.kernel,
        mesh=mesh,
        out_type=jax.ShapeDtypeStruct(
            (seq_len, d // 8, n_blocks, 8, _IB), jnp.float32
        ),
        scratch_types=[
            *[pltpu.VMEM((_IB,), jnp.int32) for _ in range(4)],
            *[pltpu.VMEM((_IB, d), jnp.float32) for _ in range(4)],
            *[pltpu.VMEM((d // 8, 8, _IB + 1), jnp.float32) for _ in range(2)],
            *[pltpu.SemaphoreType.DMA for _ in range(10)],
        ],
        compiler_params=pltpu.CompilerParams(
            use_tc_tiling_on_sc=False, needs_layout_passes=False
        ),
    )
    def gather(table_hbm, idxt_hbm, out_hbm, *scratch):
        ibufs = scratch[0:4]
        gbufs = scratch[4:8]
        tbufs = scratch[8:10]
        isems = scratch[10:14]
        gsems = scratch[14:18]
        ssems = scratch[18:20]

        wid = lax.axis_index("s") * _NC + lax.axis_index("c")

        iota = lax.iota(jnp.int32, 16)
        r_v = lax.rem(iota, 8)
        tk_base = lax.div(iota, 8)
        zeros_v = jnp.zeros((16,), jnp.int32)
        tkvs = [tk_base + 2 * q for q in range(d // 16)]

        def unit_ij(u):
            j = lax.rem(u, seq_len)
            g = lax.div(u, seq_len)
            return j, wid * blocks_per_w + g

        def issue_idx(u, b):
            j, blk = unit_ij(u)
            pltpu.async_copy(
                idxt_hbm.at[pl.ds(j * n_seq + blk * _IB, _IB)],
                ibufs[b],
                isems[b],
            )

        def wait_idx(u, b):
            j, blk = unit_ij(u)
            pltpu.make_async_copy(
                idxt_hbm.at[pl.ds(j * n_seq + blk * _IB, _IB)],
                ibufs[b],
                isems[b],
            ).wait()

        def issue_gather(b):
            pltpu.async_copy(table_hbm.at[ibufs[b]], gbufs[b], gsems[b])

        def wait_gather(b):
            pltpu.make_async_copy(
                table_hbm.at[ibufs[b]], gbufs[b], gsems[b]
            ).wait()

        def issue_scatter(u, tb):
            j, blk = unit_ij(u)
            pltpu.async_copy(
                tbufs[tb].at[:, :, pl.ds(0, _IB)],
                out_hbm.at[j, :, blk],
                ssems[tb],
            )

        def wait_scatter(u, tb):
            j, blk = unit_ij(u)
            pltpu.make_async_copy(
                tbufs[tb].at[:, :, pl.ds(0, _IB)],
                out_hbm.at[j, :, blk],
                ssems[tb],
            ).wait()

        def transpose_scale(b, tb):
            gbuf = gbufs[b]
            tbuf = tbufs[tb]

            @plsc.parallel_loop(0, _IB, 1, unroll=8)
            def _row(c):
                c_v = zeros_v + c
                for q in range(d // 16):
                    v = gbuf[c, pl.ds(16 * q, 16)] * SCALE
                    plsc.store_scatter(tbuf, [tkvs[q], r_v, c_v], v)

        for u in range(3):
            issue_idx(u, u)
        for u in range(2):
            wait_idx(u, u)
            issue_gather(u)

        def ring4(k, carry):
            for uu in range(4):
                u = k * 4 + uu

                @pl.when(u + 3 < n_units)
                def _idx():
                    issue_idx(u + 3, (uu + 3) % 4)

                @pl.when(u + 2 < n_units)
                def _gather():
                    wait_idx(u + 2, (uu + 2) % 4)
                    issue_gather((uu + 2) % 4)

                wait_gather(uu)

                @pl.when(u >= 2)
                def _drain():
                    wait_scatter(u - 2, uu % 2)

                transpose_scale(uu, uu % 2)
                issue_scatter(u, uu % 2)
            return carry

        lax.fori_loop(0, n_units // 4, ring4, 0)

        for u in range(n_units - 2, n_units):
            wait_scatter(u, u % 2)

    return gather


@jax.jit
def kernel(x, emb_weight):
    b, s = x.shape
    v, d = emb_weight.shape
    idxt = x.T.reshape(-1).astype(jnp.int32)
    out6 = _make_gather(b, s, d)(emb_weight, idxt)
    out3 = jnp.transpose(out6, (0, 1, 3, 2, 4)).reshape(s, d, b)
    return jnp.transpose(out3, (2, 0, 1))
